# baseline (speedup 1.0000x reference)
import jax
import jax.numpy as jnp
from jax import lax
from jax.experimental import pallas as pl
from jax.experimental.pallas import tpu as pltpu
from jax.experimental.pallas import tpu_sc as plsc

_D = 64
_NC = 2
_NS = 16
_NW = _NC * _NS
_CHUNK = 40
_G = 2
_ROWS_PER_W = 128


def _gather_body(idx_hbm, table_hbm, out_hbm, idx_v, bufs, g0, g1, w0, w1):
    gsem = (g0, g1)
    wsem = (w0, w1)
    wid = lax.axis_index("s") * _NC + lax.axis_index("c")
    hist = idx_hbm.shape[1]
    per_row = hist // _CHUNK
    ngrp = _ROWS_PER_W // _G
    base = wid * _ROWS_PER_W
    pltpu.sync_copy(idx_hbm.at[pl.ds(base, _ROWS_PER_W)], idx_v)

    def drain(sem, p):
        pltpu.make_async_copy(out_hbm.at[pl.ds(0, _G)], bufs.at[p], sem).wait()

    def fire_gathers(g, p):
        for r in range(_G):
            for j in range(per_row):
                pltpu.async_copy(
                    table_hbm.at[idx_v.at[g * _G + r, pl.ds(j * _CHUNK, _CHUNK)]],
                    bufs.at[p, r, pl.ds(j * _CHUNK, _CHUNK)],
                    gsem[p],
                )

    def fire_write(g, p):
        pltpu.async_copy(
            bufs.at[p], out_hbm.at[pl.ds(base + g * _G, _G)], wsem[p]
        )

    def outer(g2, carry):
        for p in range(2):
            g = g2 * 2 + p

            @pl.when(g >= 2)
            def _():
                drain(wsem[p], p)

            fire_gathers(g, p)

            @pl.when(g >= 1)
            def _():
                drain(gsem[1 - p], 1 - p)
                fire_write(g - 1, 1 - p)

        return carry

    lax.fori_loop(0, ngrp // 2, outer, 0)
    drain(gsem[1], 1)
    fire_write(ngrp - 1, 1)
    drain(wsem[0], 0)
    drain(wsem[1], 1)


def kernel(inputs, embeddings):
    b, h = inputs.shape
    assert b % (_NW * _G) == 0 and h % _CHUNK == 0
    idx = inputs.astype(jnp.int32)
    mesh = plsc.VectorSubcoreMesh(core_axis_name="c", subcore_axis_name="s")
    fn = pl.kernel(
        _gather_body,
        mesh=mesh,
        out_type=jax.ShapeDtypeStruct((b, h, _D), jnp.float32),
        scratch_types=[
            pltpu.VMEM((_ROWS_PER_W, h), jnp.int32),
            pltpu.VMEM((2, _G, h, _D), jnp.float32),
            pltpu.SemaphoreType.DMA,
            pltpu.SemaphoreType.DMA,
            pltpu.SemaphoreType.DMA,
            pltpu.SemaphoreType.DMA,
        ],
        compiler_params=pltpu.CompilerParams(use_tc_tiling_on_sc=False),
    )
    return fn(idx, embeddings)
